# Initial kernel scaffold; baseline (speedup 1.0000x reference)
import jax
import jax.numpy as jnp
from jax.experimental import pallas as pl


def kernel(x, edge_index, edge_attr, W_h, b_h, W_n, b_n, w_e, W_ft, b_ft):
    raise NotImplementedError("write your pallas kernel here")


# speedup vs baseline: 2.5460x; 2.5460x over previous
import functools

import jax
import jax.numpy as jnp
from jax import lax
from jax.experimental import pallas as pl
from jax.experimental.pallas import tpu as pltpu
from jax.experimental.pallas import tpu_sc as plsc

NC = 2
NS = 16
LANES = 16


def _pre_body(x_ref, wa_ref, wb_ref, bn_ref, xa_ref, xb_ref):
  xblk = x_ref[...]
  xa_ref[...] = jnp.dot(xblk, wa_ref[...], preferred_element_type=jnp.float32)
  xb_ref[...] = (
      jnp.dot(xblk, wb_ref[...], preferred_element_type=jnp.float32)
      + bn_ref[...]
  )


def _score_body(g_ref, ea_ref, wh_ref, bh_ref, we_ref, s_ref, m_ref):
  i = pl.program_id(0)
  h = (
      jnp.dot(ea_ref[...], wh_ref[...], preferred_element_type=jnp.float32)
      + bh_ref[...]
      + g_ref[...]
  )
  h = jnp.maximum(h, 0.0)
  s = jnp.dot(h, we_ref[...], preferred_element_type=jnp.float32)
  s_ref[...] = s

  @pl.when(i == 0)
  def _():
    m_ref[...] = jnp.full((1, 1), -jnp.inf, jnp.float32)

  m_ref[...] = jnp.maximum(m_ref[...], jnp.max(s))


def _exp_body(s_ref, m_ref, e_ref, t_ref):
  i = pl.program_id(0)
  ex = jnp.exp(s_ref[...] - m_ref[...])
  e_ref[...] = ex

  @pl.when(i == 0)
  def _():
    t_ref[...] = jnp.zeros((1, 1), jnp.float32)

  t_ref[...] = t_ref[...] + jnp.sum(ex)


def _norm_body(e_ref, t_ref, a_ref):
  a_ref[...] = e_ref[...] / t_ref[...]


def _final_body(x_ref, p0_ref, p1_ref, wft_ref, bft_ref, o_ref):
  h = x_ref[...] - p0_ref[...] - p1_ref[...]
  o_ref[...] = (
      h
      + jnp.dot(h, wft_ref[...], preferred_element_type=jnp.float32)
      + bft_ref[...]
  )


def _sc_gather_add(xa, xb, src, dst, *, n_edges, d):
  nw = NC * NS
  epw = n_edges // nw
  ch = 400
  nchunk = epw // ch
  mesh = plsc.VectorSubcoreMesh(
      core_axis_name="c", subcore_axis_name="s",
      num_cores=NC, num_subcores=NS)

  @functools.partial(
      pl.kernel,
      out_type=jax.ShapeDtypeStruct((n_edges, d), jnp.float32),
      mesh=mesh,
      scratch_types=[
          pltpu.VMEM((ch,), jnp.int32),
          pltpu.VMEM((ch,), jnp.int32),
          pltpu.VMEM((ch, d), jnp.float32),
          pltpu.VMEM((ch, d), jnp.float32),
          pltpu.SemaphoreType.DMA,
          pltpu.SemaphoreType.DMA,
      ],
  )
  def k(xa_hbm, xb_hbm, src_hbm, dst_hbm, g_hbm,
        src_v, dst_v, rows_a, rows_b, sem_a, sem_b):
    wid = lax.axis_index("s") * NC + lax.axis_index("c")
    base_w = wid * epw

    def body(j, carry):
      base = base_w + j * ch
      pltpu.sync_copy(src_hbm.at[pl.ds(base, ch)], src_v)
      pltpu.sync_copy(dst_hbm.at[pl.ds(base, ch)], dst_v)
      ca = pltpu.async_copy(xa_hbm.at[src_v], rows_a, sem_a)
      cb = pltpu.async_copy(xb_hbm.at[dst_v], rows_b, sem_b)
      ca.wait()
      cb.wait()

      def rbody(r, c2):
        for cb_ in range(d // LANES):
          sl = pl.ds(cb_ * LANES, LANES)
          rows_a[r, sl] = rows_a[r, sl] + rows_b[r, sl]
        return c2

      lax.fori_loop(0, ch, rbody, 0)
      pltpu.sync_copy(rows_a, g_hbm.at[pl.ds(base, ch)])
      return carry

    lax.fori_loop(0, nchunk, body, 0)

  return k(xa, xb, src, dst)


def _sc_scatter(x, src, dst, alpha, zeros, *, n_nodes, n_edges, d):
  nw = NC * NS
  epw = n_edges // nw
  ch = 200
  nchunk = epw // ch
  rpt = 1000
  mesh = plsc.VectorSubcoreMesh(
      core_axis_name="c", subcore_axis_name="s",
      num_cores=NC, num_subcores=NS)

  sub = 40
  nsub = ch // sub

  @functools.partial(
      pl.kernel,
      out_type=jax.ShapeDtypeStruct((NC, n_nodes, d), jnp.float32),
      mesh=mesh,
      scratch_types=[
          pltpu.VMEM((nsub, sub), jnp.int32),
          pltpu.VMEM((ch,), jnp.int32),
          pltpu.VMEM((ch,), jnp.float32),
          pltpu.VMEM((ch, d), jnp.float32),
          pltpu.SemaphoreType.DMA,
          pltpu.VMEM_SHARED((n_nodes, d), jnp.float32),
      ],
  )
  def k(x_hbm, src_hbm, dst_hbm, alpha_hbm, zeros_hbm, out_hbm,
        src_v, dst_v, alpha_v, rows_v, sem, acc):
    cid = lax.axis_index("c")
    sid = lax.axis_index("s")

    @pl.when(sid == 0)
    def _():
      pltpu.sync_copy(zeros_hbm, acc)

    plsc.subcore_barrier()

    wid = sid * NC + cid
    base_w = wid * epw

    def body(j, carry):
      base = base_w + j * ch
      for i in range(nsub):
        pltpu.sync_copy(src_hbm.at[pl.ds(base + i * sub, sub)], src_v.at[i])
      pltpu.sync_copy(dst_hbm.at[pl.ds(base, ch)], dst_v)
      pltpu.sync_copy(alpha_hbm.at[pl.ds(base, ch)], alpha_v)
      pltpu.async_copy(x_hbm.at[dst_v], rows_v, sem).wait()

      def rbody(grp, c2):
        a16 = alpha_v[pl.ds(grp * LANES, LANES)]
        for rr in range(LANES):
          r = grp * LANES + rr
          a = a16[rr]
          for cb in range(d // LANES):
            sl = pl.ds(cb * LANES, LANES)
            rows_v[r, sl] = rows_v[r, sl] * a
        return c2

      lax.fori_loop(0, ch // LANES, rbody, 0)
      ntail = ch - (ch // LANES) * LANES
      if ntail:
        a16 = alpha_v[pl.ds(ch - LANES, LANES)]
        for rr in range(ntail):
          r = (ch // LANES) * LANES + rr
          a = a16[LANES - ntail + rr]
          for cb in range(d // LANES):
            sl = pl.ds(cb * LANES, LANES)
            rows_v[r, sl] = rows_v[r, sl] * a
      for i in range(nsub):
        pltpu.sync_copy(rows_v.at[pl.ds(i * sub, sub)],
                        acc.at[src_v.at[i]], add=True)
      return carry

    lax.fori_loop(0, nchunk, body, 0)
    plsc.subcore_barrier()

    @pl.when(sid < n_nodes // rpt)
    def _():
      pltpu.sync_copy(acc.at[pl.ds(sid * rpt, rpt)],
                      out_hbm.at[cid, pl.ds(sid * rpt, rpt)])

  return k(x, src, dst, alpha, zeros)


def kernel(x, edge_index, edge_attr, W_h, b_h, W_n, b_n, w_e, W_ft, b_ft):
  n, d = x.shape
  e = edge_index.shape[1]
  src = edge_index[0]
  dst = edge_index[1]

  wa = W_n[:, :d].T
  wb = W_n[:, d:].T
  nb = 2000
  xa, xb = pl.pallas_call(
      _pre_body,
      grid=(n // nb,),
      in_specs=[
          pl.BlockSpec((nb, d), lambda i: (i, 0)),
          pl.BlockSpec(wa.shape, lambda i: (0, 0)),
          pl.BlockSpec(wb.shape, lambda i: (0, 0)),
          pl.BlockSpec((1, d), lambda i: (0, 0)),
      ],
      out_specs=[
          pl.BlockSpec((nb, d), lambda i: (i, 0)),
          pl.BlockSpec((nb, d), lambda i: (i, 0)),
      ],
      out_shape=[
          jax.ShapeDtypeStruct((n, d), jnp.float32),
          jax.ShapeDtypeStruct((n, d), jnp.float32),
      ],
  )(x, wa, wb, b_n.reshape(1, d))

  g = _sc_gather_add(xa, xb, src, dst, n_edges=e, d=d)

  be = 3200
  grid_e = e // be
  ed = edge_attr.shape[1]
  scores, gmax = pl.pallas_call(
      _score_body,
      grid=(grid_e,),
      in_specs=[
          pl.BlockSpec((be, d), lambda i: (i, 0)),
          pl.BlockSpec((be, ed), lambda i: (i, 0)),
          pl.BlockSpec((ed, d), lambda i: (0, 0)),
          pl.BlockSpec((1, d), lambda i: (0, 0)),
          pl.BlockSpec((d, 1), lambda i: (0, 0)),
      ],
      out_specs=[
          pl.BlockSpec((be, 1), lambda i: (i, 0)),
          pl.BlockSpec((1, 1), lambda i: (0, 0)),
      ],
      out_shape=[
          jax.ShapeDtypeStruct((e, 1), jnp.float32),
          jax.ShapeDtypeStruct((1, 1), jnp.float32),
      ],
  )(g, edge_attr, W_h.T, b_h.reshape(1, d), w_e)

  expv, tot = pl.pallas_call(
      _exp_body,
      grid=(grid_e,),
      in_specs=[
          pl.BlockSpec((be, 1), lambda i: (i, 0)),
          pl.BlockSpec((1, 1), lambda i: (0, 0)),
      ],
      out_specs=[
          pl.BlockSpec((be, 1), lambda i: (i, 0)),
          pl.BlockSpec((1, 1), lambda i: (0, 0)),
      ],
      out_shape=[
          jax.ShapeDtypeStruct((e, 1), jnp.float32),
          jax.ShapeDtypeStruct((1, 1), jnp.float32),
      ],
  )(scores, gmax)

  alpha2d = pl.pallas_call(
      _norm_body,
      grid=(grid_e,),
      in_specs=[
          pl.BlockSpec((be, 1), lambda i: (i, 0)),
          pl.BlockSpec((1, 1), lambda i: (0, 0)),
      ],
      out_specs=pl.BlockSpec((be, 1), lambda i: (i, 0)),
      out_shape=jax.ShapeDtypeStruct((e, 1), jnp.float32),
  )(expv, tot)
  alpha = alpha2d.reshape(e)

  zeros = jnp.zeros((n, d), jnp.float32)
  partials = _sc_scatter(x, src, dst, alpha, zeros,
                         n_nodes=n, n_edges=e, d=d)

  out = pl.pallas_call(
      _final_body,
      grid=(n // nb,),
      in_specs=[
          pl.BlockSpec((nb, d), lambda i: (i, 0)),
          pl.BlockSpec((nb, d), lambda i: (i, 0)),
          pl.BlockSpec((nb, d), lambda i: (i, 0)),
          pl.BlockSpec((d, d), lambda i: (0, 0)),
          pl.BlockSpec((1, d), lambda i: (0, 0)),
      ],
      out_specs=pl.BlockSpec((nb, d), lambda i: (i, 0)),
      out_shape=jax.ShapeDtypeStruct((n, d), jnp.float32),
  )(x, partials[0], partials[1], W_ft.T, b_ft.reshape(1, d))

  return (out, alpha)


# speedup vs baseline: 3.3290x; 1.3075x over previous
import functools

import jax
import jax.numpy as jnp
from jax import lax
from jax.experimental import pallas as pl
from jax.experimental.pallas import tpu as pltpu
from jax.experimental.pallas import tpu_sc as plsc

NC = 2
NS = 16
LANES = 16


def _pre_body(x_ref, wa_ref, wb_ref, bn_ref, xa_ref, xb_ref):
  xblk = x_ref[...]
  xa_ref[...] = jnp.dot(xblk, wa_ref[...], preferred_element_type=jnp.float32)
  xb_ref[...] = (
      jnp.dot(xblk, wb_ref[...], preferred_element_type=jnp.float32)
      + bn_ref[...]
  )


def _score_body(g_ref, ea_ref, wh_ref, bh_ref, we_ref, s_ref, m_ref):
  i = pl.program_id(0)
  be = g_ref.shape[0]
  h = (
      jnp.dot(ea_ref[...], wh_ref[...], preferred_element_type=jnp.float32)
      + bh_ref[...]
      + g_ref[...]
  )
  h = jnp.maximum(h, 0.0)
  s = jnp.sum(h * we_ref[...], axis=1)
  s_ref[pl.ds(i * be, be)] = s

  @pl.when(i == 0)
  def _():
    m_ref[...] = jnp.full((1, 1), -jnp.inf, jnp.float32)

  m_ref[...] = jnp.maximum(m_ref[...], jnp.max(s))


def _softmax_body(s_ref, m_ref, a_ref):
  ex = jnp.exp(s_ref[...] - m_ref[0, 0])
  a_ref[...] = ex / jnp.sum(ex)


def _final_body(x_ref, p0_ref, p1_ref, wft_ref, bft_ref, o_ref):
  h = x_ref[...] - p0_ref[...] - p1_ref[...]
  o_ref[...] = (
      h
      + jnp.dot(h, wft_ref[...], preferred_element_type=jnp.float32)
      + bft_ref[...]
  )


def _sc_gather_add(xa, xb, src, dst, *, n_edges, d):
  nw = NC * NS
  epw = n_edges // nw
  ch = 400
  nchunk = epw // ch
  mesh = plsc.VectorSubcoreMesh(
      core_axis_name="c", subcore_axis_name="s",
      num_cores=NC, num_subcores=NS)

  @functools.partial(
      pl.kernel,
      out_type=jax.ShapeDtypeStruct((n_edges, d), jnp.float32),
      mesh=mesh,
      scratch_types=[
          pltpu.VMEM((ch,), jnp.int32),
          pltpu.VMEM((ch,), jnp.int32),
          pltpu.VMEM((ch, d), jnp.float32),
          pltpu.VMEM((ch, d), jnp.float32),
          pltpu.SemaphoreType.DMA,
          pltpu.SemaphoreType.DMA,
      ],
  )
  def k(xa_hbm, xb_hbm, src_hbm, dst_hbm, g_hbm,
        src_v, dst_v, rows_a, rows_b, sem_a, sem_b):
    wid = lax.axis_index("s") * NC + lax.axis_index("c")
    base_w = wid * epw

    def body(j, carry):
      base = base_w + j * ch
      pltpu.sync_copy(src_hbm.at[pl.ds(base, ch)], src_v)
      pltpu.sync_copy(dst_hbm.at[pl.ds(base, ch)], dst_v)
      ca = pltpu.async_copy(xa_hbm.at[src_v], rows_a, sem_a)
      cb = pltpu.async_copy(xb_hbm.at[dst_v], rows_b, sem_b)
      ca.wait()
      cb.wait()

      def rbody(r, c2):
        for cb_ in range(d // LANES):
          sl = pl.ds(cb_ * LANES, LANES)
          rows_a[r, sl] = rows_a[r, sl] + rows_b[r, sl]
        return c2

      lax.fori_loop(0, ch, rbody, 0)
      pltpu.sync_copy(rows_a, g_hbm.at[pl.ds(base, ch)])
      return carry

    lax.fori_loop(0, nchunk, body, 0)

  return k(xa, xb, src, dst)


def _sc_scatter(x, src, dst, alpha, zeros, *, n_nodes, n_edges, d):
  nw = NC * NS
  epw = n_edges // nw
  ch = 200
  nchunk = epw // ch
  rpt = 1000
  mesh = plsc.VectorSubcoreMesh(
      core_axis_name="c", subcore_axis_name="s",
      num_cores=NC, num_subcores=NS)

  sub = 40
  nsub = ch // sub

  @functools.partial(
      pl.kernel,
      out_type=jax.ShapeDtypeStruct((NC, n_nodes, d), jnp.float32),
      mesh=mesh,
      scratch_types=[
          pltpu.VMEM((nsub, sub), jnp.int32),
          pltpu.VMEM((ch,), jnp.int32),
          pltpu.VMEM((ch,), jnp.float32),
          pltpu.VMEM((ch, d), jnp.float32),
          pltpu.SemaphoreType.DMA,
          pltpu.VMEM_SHARED((n_nodes, d), jnp.float32),
      ],
  )
  def k(x_hbm, src_hbm, dst_hbm, alpha_hbm, zeros_hbm, out_hbm,
        src_v, dst_v, alpha_v, rows_v, sem, acc):
    cid = lax.axis_index("c")
    sid = lax.axis_index("s")

    @pl.when(sid == 0)
    def _():
      pltpu.sync_copy(zeros_hbm, acc)

    plsc.subcore_barrier()

    wid = sid * NC + cid
    base_w = wid * epw

    def body(j, carry):
      base = base_w + j * ch
      for i in range(nsub):
        pltpu.sync_copy(src_hbm.at[pl.ds(base + i * sub, sub)], src_v.at[i])
      pltpu.sync_copy(dst_hbm.at[pl.ds(base, ch)], dst_v)
      pltpu.sync_copy(alpha_hbm.at[pl.ds(base, ch)], alpha_v)
      pltpu.async_copy(x_hbm.at[dst_v], rows_v, sem).wait()

      def rbody(grp, c2):
        a16 = alpha_v[pl.ds(grp * LANES, LANES)]
        for rr in range(LANES):
          r = grp * LANES + rr
          a = a16[rr]
          for cb in range(d // LANES):
            sl = pl.ds(cb * LANES, LANES)
            rows_v[r, sl] = rows_v[r, sl] * a
        return c2

      lax.fori_loop(0, ch // LANES, rbody, 0)
      ntail = ch - (ch // LANES) * LANES
      if ntail:
        a16 = alpha_v[pl.ds(ch - LANES, LANES)]
        for rr in range(ntail):
          r = (ch // LANES) * LANES + rr
          a = a16[LANES - ntail + rr]
          for cb in range(d // LANES):
            sl = pl.ds(cb * LANES, LANES)
            rows_v[r, sl] = rows_v[r, sl] * a
      for i in range(nsub):
        pltpu.sync_copy(rows_v.at[pl.ds(i * sub, sub)],
                        acc.at[src_v.at[i]], add=True)
      return carry

    lax.fori_loop(0, nchunk, body, 0)
    plsc.subcore_barrier()

    @pl.when(sid < n_nodes // rpt)
    def _():
      pltpu.sync_copy(acc.at[pl.ds(sid * rpt, rpt)],
                      out_hbm.at[cid, pl.ds(sid * rpt, rpt)])

  return k(x, src, dst, alpha, zeros)


def kernel(x, edge_index, edge_attr, W_h, b_h, W_n, b_n, w_e, W_ft, b_ft):
  n, d = x.shape
  e = edge_index.shape[1]
  src = edge_index[0]
  dst = edge_index[1]

  wa = W_n[:, :d].T
  wb = W_n[:, d:].T
  nb = 2000
  xa, xb = pl.pallas_call(
      _pre_body,
      grid=(n // nb,),
      in_specs=[
          pl.BlockSpec((nb, d), lambda i: (i, 0)),
          pl.BlockSpec(wa.shape, lambda i: (0, 0)),
          pl.BlockSpec(wb.shape, lambda i: (0, 0)),
          pl.BlockSpec((1, d), lambda i: (0, 0)),
      ],
      out_specs=[
          pl.BlockSpec((nb, d), lambda i: (i, 0)),
          pl.BlockSpec((nb, d), lambda i: (i, 0)),
      ],
      out_shape=[
          jax.ShapeDtypeStruct((n, d), jnp.float32),
          jax.ShapeDtypeStruct((n, d), jnp.float32),
      ],
  )(x, wa, wb, b_n.reshape(1, d))

  g = _sc_gather_add(xa, xb, src, dst, n_edges=e, d=d)

  be = 3200
  grid_e = e // be
  ed = edge_attr.shape[1]
  scores, gmax = pl.pallas_call(
      _score_body,
      grid=(grid_e,),
      in_specs=[
          pl.BlockSpec((be, d), lambda i: (i, 0)),
          pl.BlockSpec((be, ed), lambda i: (i, 0)),
          pl.BlockSpec((ed, d), lambda i: (0, 0)),
          pl.BlockSpec((1, d), lambda i: (0, 0)),
          pl.BlockSpec((1, d), lambda i: (0, 0)),
      ],
      out_specs=[
          pl.BlockSpec((e,), lambda i: (0,)),
          pl.BlockSpec((1, 1), lambda i: (0, 0)),
      ],
      out_shape=[
          jax.ShapeDtypeStruct((e,), jnp.float32),
          jax.ShapeDtypeStruct((1, 1), jnp.float32),
      ],
  )(g, edge_attr, W_h.T, b_h.reshape(1, d), w_e.reshape(1, d))

  alpha = pl.pallas_call(
      _softmax_body,
      in_specs=[
          pl.BlockSpec((e,), lambda: (0,)),
          pl.BlockSpec((1, 1), lambda: (0, 0)),
      ],
      out_specs=pl.BlockSpec((e,), lambda: (0,)),
      out_shape=jax.ShapeDtypeStruct((e,), jnp.float32),
  )(scores, gmax)

  zeros = jnp.zeros((n, d), jnp.float32)
  partials = _sc_scatter(x, src, dst, alpha, zeros,
                         n_nodes=n, n_edges=e, d=d)

  out = pl.pallas_call(
      _final_body,
      grid=(n // nb,),
      in_specs=[
          pl.BlockSpec((nb, d), lambda i: (i, 0)),
          pl.BlockSpec((nb, d), lambda i: (i, 0)),
          pl.BlockSpec((nb, d), lambda i: (i, 0)),
          pl.BlockSpec((d, d), lambda i: (0, 0)),
          pl.BlockSpec((1, d), lambda i: (0, 0)),
      ],
      out_specs=pl.BlockSpec((nb, d), lambda i: (i, 0)),
      out_shape=jax.ShapeDtypeStruct((n, d), jnp.float32),
  )(x, partials[0], partials[1], W_ft.T, b_ft.reshape(1, d))

  return (out, alpha)


# speedup vs baseline: 3.8203x; 1.1476x over previous
import functools

import jax
import jax.numpy as jnp
from jax import lax
from jax.experimental import pallas as pl
from jax.experimental.pallas import tpu as pltpu
from jax.experimental.pallas import tpu_sc as plsc

NC = 2
NS = 16
LANES = 16


def _pre_body(x_ref, wa_ref, wb_ref, bn_ref, xa_ref, xb_ref):
  xblk = x_ref[...]
  xa_ref[...] = jnp.dot(xblk, wa_ref[...], preferred_element_type=jnp.float32)
  xb_ref[...] = (
      jnp.dot(xblk, wb_ref[...], preferred_element_type=jnp.float32)
      + bn_ref[...]
  )


def _score_body(g_ref, ea_ref, wh_ref, bh_ref, we_ref, s_ref, m_ref):
  i = pl.program_id(0)
  be = g_ref.shape[0]
  h = (
      jnp.dot(ea_ref[...], wh_ref[...], preferred_element_type=jnp.float32)
      + bh_ref[...]
      + g_ref[...]
  )
  h = jnp.maximum(h, 0.0)
  s = jnp.sum(h * we_ref[...], axis=1)
  s_ref[pl.ds(i * be, be)] = s

  @pl.when(i == 0)
  def _():
    m_ref[...] = jnp.full((1, 1), -jnp.inf, jnp.float32)

  m_ref[...] = jnp.maximum(m_ref[...], jnp.max(s))


def _softmax_body(s_ref, m_ref, a_ref):
  ex = jnp.exp(s_ref[...] - m_ref[0, 0])
  a_ref[...] = ex / jnp.sum(ex)


def _final_body(x_ref, p0_ref, p1_ref, wft_ref, bft_ref, o_ref):
  h = (x_ref[...] - p0_ref[...].astype(jnp.float32)
       - p1_ref[...].astype(jnp.float32))
  o_ref[...] = (
      h
      + jnp.dot(h, wft_ref[...], preferred_element_type=jnp.float32)
      + bft_ref[...]
  )


def _sc_gather_add(xa, xb, src, dst, *, n_edges, d):
  nw = NC * NS
  epw = n_edges // nw
  ch = 200
  nchunk = epw // ch
  npair = nchunk // 2
  mesh = plsc.VectorSubcoreMesh(
      core_axis_name="c", subcore_axis_name="s",
      num_cores=NC, num_subcores=NS)

  @functools.partial(
      pl.kernel,
      out_type=jax.ShapeDtypeStruct((n_edges, d), jnp.float32),
      mesh=mesh,
      scratch_types=[
          pltpu.VMEM((ch,), jnp.int32),
          pltpu.VMEM((ch,), jnp.int32),
          pltpu.VMEM((ch,), jnp.int32),
          pltpu.VMEM((ch,), jnp.int32),
          pltpu.VMEM((ch, d), jnp.float32),
          pltpu.VMEM((ch, d), jnp.float32),
          pltpu.VMEM((ch, d), jnp.float32),
          pltpu.VMEM((ch, d), jnp.float32),
          pltpu.SemaphoreType.DMA,
          pltpu.SemaphoreType.DMA,
          pltpu.SemaphoreType.DMA,
          pltpu.SemaphoreType.DMA,
      ],
  )
  def k(xa_hbm, xb_hbm, src_hbm, dst_hbm, g_hbm,
        src0, src1, dst0, dst1, ra0, ra1, rb0, rb1, sa0, sa1, sb0, sb1):
    wid = lax.axis_index("s") * NC + lax.axis_index("c")
    base_w = wid * epw
    srcs = (src0, src1)
    dsts = (dst0, dst1)
    ras = (ra0, ra1)
    rbs = (rb0, rb1)
    sems_a = (sa0, sa1)
    sems_b = (sb0, sb1)

    def start(c, b):
      base = base_w + c * ch
      pltpu.sync_copy(src_hbm.at[pl.ds(base, ch)], srcs[b])
      pltpu.sync_copy(dst_hbm.at[pl.ds(base, ch)], dsts[b])
      pltpu.async_copy(xa_hbm.at[srcs[b]], ras[b], sems_a[b])
      pltpu.async_copy(xb_hbm.at[dsts[b]], rbs[b], sems_b[b])

    def process(c, b):
      pltpu.make_async_copy(xa_hbm.at[srcs[b]], ras[b], sems_a[b]).wait()
      pltpu.make_async_copy(xb_hbm.at[dsts[b]], rbs[b], sems_b[b]).wait()
      ra, rb = ras[b], rbs[b]

      def rbody(r, c2):
        for cb_ in range(d // LANES):
          sl = pl.ds(cb_ * LANES, LANES)
          ra[r, sl] = ra[r, sl] + rb[r, sl]
        return c2

      lax.fori_loop(0, ch, rbody, 0)
      pltpu.sync_copy(ra, g_hbm.at[pl.ds(base_w + c * ch, ch)])

    start(0, 0)

    def body(g, carry):
      c0 = 2 * g
      start(c0 + 1, 1)
      process(c0, 0)

      @pl.when(g < npair - 1)
      def _():
        start(c0 + 2, 0)

      process(c0 + 1, 1)
      return carry

    lax.fori_loop(0, npair, body, 0)

  return k(xa, xb, src, dst)


def _sc_scatter(x, src, dst, alpha, zeros, *, n_nodes, n_edges, d):
  nw = NC * NS
  epw = n_edges // nw
  ch = 80
  nchunk = epw // ch
  npair = nchunk // 2
  rpt = 1000
  mesh = plsc.VectorSubcoreMesh(
      core_axis_name="c", subcore_axis_name="s",
      num_cores=NC, num_subcores=NS)

  sub = 40
  nsub = ch // sub

  @functools.partial(
      pl.kernel,
      out_type=jax.ShapeDtypeStruct((NC, n_nodes, d), jnp.float32),
      mesh=mesh,
      scratch_types=[
          pltpu.VMEM((nsub, sub), jnp.int32),
          pltpu.VMEM((nsub, sub), jnp.int32),
          pltpu.VMEM((ch,), jnp.int32),
          pltpu.VMEM((ch,), jnp.int32),
          pltpu.VMEM((ch,), jnp.float32),
          pltpu.VMEM((ch,), jnp.float32),
          pltpu.VMEM((ch, d), jnp.float32),
          pltpu.VMEM((ch, d), jnp.float32),
          pltpu.SemaphoreType.DMA,
          pltpu.SemaphoreType.DMA,
          pltpu.SemaphoreType.DMA,
          pltpu.SemaphoreType.DMA,
          pltpu.VMEM_SHARED((n_nodes, d), jnp.float32),
      ],
  )
  def k(x_hbm, src_hbm, dst_hbm, alpha_hbm, zeros_hbm, out_hbm,
        sv0, sv1, dv0, dv1, av0, av1, rv0, rv1, sg0, sg1, ss0, ss1, acc):
    cid = lax.axis_index("c")
    sid = lax.axis_index("s")
    srcs = (sv0, sv1)
    dsts = (dv0, dv1)
    alphas = (av0, av1)
    rows = (rv0, rv1)
    sems_g = (sg0, sg1)
    sems_s = (ss0, ss1)

    @pl.when(sid == 0)
    def _():
      pltpu.sync_copy(zeros_hbm, acc)

    plsc.subcore_barrier()

    wid = sid * NC + cid
    base_w = wid * epw

    def start(c, b):
      base = base_w + c * ch
      for i in range(nsub):
        pltpu.sync_copy(src_hbm.at[pl.ds(base + i * sub, sub)],
                        srcs[b].at[i])
      pltpu.sync_copy(dst_hbm.at[pl.ds(base, ch)], dsts[b])
      pltpu.sync_copy(alpha_hbm.at[pl.ds(base, ch)], alphas[b])
      pltpu.async_copy(x_hbm.at[dsts[b]], rows[b], sems_g[b])

    def wait_scatters(b):
      for i in range(nsub):
        pltpu.make_async_copy(rows[b].at[pl.ds(i * sub, sub)],
                              acc.at[srcs[b].at[i]], sems_s[b]).wait()

    def process(c, b):
      pltpu.make_async_copy(x_hbm.at[dsts[b]], rows[b], sems_g[b]).wait()
      alpha_v, rows_v = alphas[b], rows[b]

      def rbody(grp, c2):
        a16 = alpha_v[pl.ds(grp * LANES, LANES)]
        for rr in range(LANES):
          r = grp * LANES + rr
          a = a16[rr]
          for cb in range(d // LANES):
            sl = pl.ds(cb * LANES, LANES)
            rows_v[r, sl] = rows_v[r, sl] * a
        return c2

      lax.fori_loop(0, ch // LANES, rbody, 0)
      for i in range(nsub):
        pltpu.async_copy(rows_v.at[pl.ds(i * sub, sub)],
                         acc.at[srcs[b].at[i]], sems_s[b], add=True)

    start(0, 0)

    def body(g, carry):
      c0 = 2 * g

      @pl.when(g > 0)
      def _():
        wait_scatters(1)

      start(c0 + 1, 1)
      process(c0, 0)

      @pl.when(g < npair - 1)
      def _():
        wait_scatters(0)
        start(c0 + 2, 0)

      process(c0 + 1, 1)
      return carry

    lax.fori_loop(0, npair, body, 0)
    if nchunk % 2:
      wait_scatters(0)
      start(nchunk - 1, 0)
      process(nchunk - 1, 0)
    wait_scatters(0)
    wait_scatters(1)
    plsc.subcore_barrier()

    @pl.when(sid < n_nodes // rpt)
    def _():
      pltpu.sync_copy(acc.at[pl.ds(sid * rpt, rpt)],
                      out_hbm.at[cid, pl.ds(sid * rpt, rpt)])

  return k(x, src, dst, alpha, zeros)


def kernel(x, edge_index, edge_attr, W_h, b_h, W_n, b_n, w_e, W_ft, b_ft):
  n, d = x.shape
  e = edge_index.shape[1]
  src = edge_index[0]
  dst = edge_index[1]

  wa = W_n[:, :d].T
  wb = W_n[:, d:].T
  nb = 2000
  xa, xb = pl.pallas_call(
      _pre_body,
      grid=(n // nb,),
      in_specs=[
          pl.BlockSpec((nb, d), lambda i: (i, 0)),
          pl.BlockSpec(wa.shape, lambda i: (0, 0)),
          pl.BlockSpec(wb.shape, lambda i: (0, 0)),
          pl.BlockSpec((1, d), lambda i: (0, 0)),
      ],
      out_specs=[
          pl.BlockSpec((nb, d), lambda i: (i, 0)),
          pl.BlockSpec((nb, d), lambda i: (i, 0)),
      ],
      out_shape=[
          jax.ShapeDtypeStruct((n, d), jnp.float32),
          jax.ShapeDtypeStruct((n, d), jnp.float32),
      ],
  )(x, wa, wb, b_n.reshape(1, d))

  g = _sc_gather_add(xa, xb, src, dst, n_edges=e, d=d)

  be = 16000
  grid_e = e // be
  ed = edge_attr.shape[1]
  scores, gmax = pl.pallas_call(
      _score_body,
      grid=(grid_e,),
      in_specs=[
          pl.BlockSpec((be, d), lambda i: (i, 0)),
          pl.BlockSpec((be, ed), lambda i: (i, 0)),
          pl.BlockSpec((ed, d), lambda i: (0, 0)),
          pl.BlockSpec((1, d), lambda i: (0, 0)),
          pl.BlockSpec((1, d), lambda i: (0, 0)),
      ],
      out_specs=[
          pl.BlockSpec((e,), lambda i: (0,)),
          pl.BlockSpec((1, 1), lambda i: (0, 0)),
      ],
      out_shape=[
          jax.ShapeDtypeStruct((e,), jnp.float32),
          jax.ShapeDtypeStruct((1, 1), jnp.float32),
      ],
  )(g, edge_attr, W_h.T, b_h.reshape(1, d), w_e.reshape(1, d))

  alpha = pl.pallas_call(
      _softmax_body,
      in_specs=[
          pl.BlockSpec((e,), lambda: (0,)),
          pl.BlockSpec((1, 1), lambda: (0, 0)),
      ],
      out_specs=pl.BlockSpec((e,), lambda: (0,)),
      out_shape=jax.ShapeDtypeStruct((e,), jnp.float32),
  )(scores, gmax)

  zeros = jnp.zeros((n, d), jnp.float32)
  partials = _sc_scatter(x, src, dst, alpha, zeros,
                         n_nodes=n, n_edges=e, d=d)

  out = pl.pallas_call(
      _final_body,
      grid=(n // nb,),
      in_specs=[
          pl.BlockSpec((nb, d), lambda i: (i, 0)),
          pl.BlockSpec((nb, d), lambda i: (i, 0)),
          pl.BlockSpec((nb, d), lambda i: (i, 0)),
          pl.BlockSpec((d, d), lambda i: (0, 0)),
          pl.BlockSpec((1, d), lambda i: (0, 0)),
      ],
      out_specs=pl.BlockSpec((nb, d), lambda i: (i, 0)),
      out_shape=jax.ShapeDtypeStruct((n, d), jnp.float32),
  )(x, partials[0], partials[1], W_ft.T, b_ft.reshape(1, d))

  return (out, alpha)


# speedup vs baseline: 5.4343x; 1.4225x over previous
import functools

import jax
import jax.numpy as jnp
from jax import lax
from jax.experimental import pallas as pl
from jax.experimental.pallas import tpu as pltpu
from jax.experimental.pallas import tpu_sc as plsc

NC = 2
NS = 16
LANES = 16


def _pre_body(x_ref, wa_ref, wb_ref, bn_ref, xa_ref, xb_ref):
  xblk = x_ref[...]
  xa_ref[...] = jnp.dot(xblk, wa_ref[...], preferred_element_type=jnp.float32)
  xb_ref[...] = (
      jnp.dot(xblk, wb_ref[...], preferred_element_type=jnp.float32)
      + bn_ref[...]
  )


def _score_body(g_ref, ea_ref, wh_ref, bh_ref, we_ref, s_ref, m_ref):
  i = pl.program_id(0)
  be = g_ref.shape[0]
  h = (
      lax.dot_general(ea_ref[...], wh_ref[...],
                      dimension_numbers=(((0,), (0,)), ((), ())),
                      preferred_element_type=jnp.float32)
      + bh_ref[...]
      + g_ref[...]
  )
  h = jnp.maximum(h, 0.0)
  s = jnp.sum(h * we_ref[...], axis=1)
  s_ref[pl.ds(i * be, be)] = s

  @pl.when(i == 0)
  def _():
    m_ref[...] = jnp.full((1, 1), -jnp.inf, jnp.float32)

  m_ref[...] = jnp.maximum(m_ref[...], jnp.max(s))


def _softmax_body(s_ref, m_ref, a_ref):
  ex = jnp.exp(s_ref[...] - m_ref[0, 0])
  a_ref[...] = ex / jnp.sum(ex)


def _final_body(x_ref, p0_ref, p1_ref, wft_ref, bft_ref, o_ref):
  h = (x_ref[...] - p0_ref[...].astype(jnp.float32)
       - p1_ref[...].astype(jnp.float32))
  o_ref[...] = (
      h
      + jnp.dot(h, wft_ref[...], preferred_element_type=jnp.float32)
      + bft_ref[...]
  )


def _sc_gather_add(xa, xb, src, dst, *, n_edges, d):
  nw = NC * NS
  epw = n_edges // nw
  ch = 200
  nchunk = epw // ch
  npair = nchunk // 2
  mesh = plsc.VectorSubcoreMesh(
      core_axis_name="c", subcore_axis_name="s",
      num_cores=NC, num_subcores=NS)

  @functools.partial(
      pl.kernel,
      out_type=jax.ShapeDtypeStruct((n_edges, d), jnp.float32),
      mesh=mesh,
      scratch_types=[
          pltpu.VMEM((epw,), jnp.int32),
          pltpu.VMEM((epw,), jnp.int32),
          pltpu.VMEM((ch, d), jnp.float32),
          pltpu.VMEM((ch, d), jnp.float32),
          pltpu.VMEM((ch, d), jnp.float32),
          pltpu.VMEM((ch, d), jnp.float32),
          pltpu.SemaphoreType.DMA,
          pltpu.SemaphoreType.DMA,
          pltpu.SemaphoreType.DMA,
          pltpu.SemaphoreType.DMA,
      ],
  )
  def k(xa_hbm, xb_hbm, src_hbm, dst_hbm, g_hbm,
        src_f, dst_f, ra0, ra1, rb0, rb1, sa0, sa1, sb0, sb1):
    wid = lax.axis_index("s") * NC + lax.axis_index("c")
    base_w = wid * epw
    ras = (ra0, ra1)
    rbs = (rb0, rb1)
    sems_a = (sa0, sa1)
    sems_b = (sb0, sb1)

    pltpu.sync_copy(src_hbm.at[pl.ds(base_w, epw)], src_f)
    pltpu.sync_copy(dst_hbm.at[pl.ds(base_w, epw)], dst_f)

    def start(c, b):
      pltpu.async_copy(xa_hbm.at[src_f.at[pl.ds(c * ch, ch)]],
                       ras[b], sems_a[b])
      pltpu.async_copy(xb_hbm.at[dst_f.at[pl.ds(c * ch, ch)]],
                       rbs[b], sems_b[b])

    def process(c, b):
      pltpu.make_async_copy(xa_hbm.at[src_f.at[pl.ds(c * ch, ch)]],
                            ras[b], sems_a[b]).wait()
      pltpu.make_async_copy(xb_hbm.at[dst_f.at[pl.ds(c * ch, ch)]],
                            rbs[b], sems_b[b]).wait()
      ra, rb = ras[b], rbs[b]

      def rbody(r, c2):
        for cb_ in range(d // LANES):
          sl = pl.ds(cb_ * LANES, LANES)
          ra[r, sl] = ra[r, sl] + rb[r, sl]
        return c2

      lax.fori_loop(0, ch, rbody, 0)
      pltpu.sync_copy(ra, g_hbm.at[pl.ds(base_w + c * ch, ch)])

    start(0, 0)

    def body(g, carry):
      c0 = 2 * g
      start(c0 + 1, 1)
      process(c0, 0)

      @pl.when(g < npair - 1)
      def _():
        start(c0 + 2, 0)

      process(c0 + 1, 1)
      return carry

    lax.fori_loop(0, npair, body, 0)

  return k(xa, xb, src, dst)


def _sc_scatter(x, src, dst, alpha, zeros, *, n_nodes, n_edges, d):
  nw = NC * NS
  epw = n_edges // nw
  ch = 80
  nchunk = epw // ch
  npair = nchunk // 2
  rpt = 1000
  mesh = plsc.VectorSubcoreMesh(
      core_axis_name="c", subcore_axis_name="s",
      num_cores=NC, num_subcores=NS)

  @functools.partial(
      pl.kernel,
      out_type=jax.ShapeDtypeStruct((NC, n_nodes, d), jnp.float32),
      mesh=mesh,
      scratch_types=[
          pltpu.VMEM((epw,), jnp.int32),
          pltpu.VMEM((epw,), jnp.float32),
          pltpu.VMEM((ch,), jnp.int32),
          pltpu.VMEM((ch,), jnp.int32),
          pltpu.VMEM((ch, d), jnp.float32),
          pltpu.VMEM((ch, d), jnp.float32),
          pltpu.SemaphoreType.DMA,
          pltpu.SemaphoreType.DMA,
          pltpu.SemaphoreType.DMA,
          pltpu.SemaphoreType.DMA,
          pltpu.SemaphoreType.DMA,
          pltpu.SemaphoreType.DMA,
          pltpu.VMEM_SHARED((n_nodes, d), jnp.float32),
      ],
  )
  def k(x_hbm, src_hbm, dst_hbm, alpha_hbm, zeros_hbm, out_hbm,
        dst_f, alpha_f, sv0, sv1, rv0, rv1,
        sg0, sg1, ss0, ss1, si0, si1, acc):
    cid = lax.axis_index("c")
    sid = lax.axis_index("s")
    srcs = (sv0, sv1)
    rows = (rv0, rv1)
    sems_g = (sg0, sg1)
    sems_s = (ss0, ss1)
    sems_i = (si0, si1)

    @pl.when(sid == 0)
    def _():
      pltpu.sync_copy(zeros_hbm, acc)

    plsc.subcore_barrier()

    wid = sid * NC + cid
    base_w = wid * epw

    pltpu.sync_copy(dst_hbm.at[pl.ds(base_w, epw)], dst_f)
    pltpu.sync_copy(alpha_hbm.at[pl.ds(base_w, epw)], alpha_f)

    def start(c, b):
      pltpu.async_copy(src_hbm.at[pl.ds(base_w + c * ch, ch)],
                       srcs[b], sems_i[b])
      pltpu.async_copy(x_hbm.at[dst_f.at[pl.ds(c * ch, ch)]],
                       rows[b], sems_g[b])

    def wait_scatters(b):
      pltpu.make_async_copy(rows[b], acc.at[srcs[b]], sems_s[b]).wait()

    def process(c, b):
      pltpu.make_async_copy(x_hbm.at[dst_f.at[pl.ds(c * ch, ch)]],
                            rows[b], sems_g[b]).wait()
      rows_v = rows[b]

      def rbody(grp, c2):
        a16 = alpha_f[pl.ds(c * ch + grp * LANES, LANES)]
        for rr in range(LANES):
          r = grp * LANES + rr
          a = a16[rr]
          for cb in range(d // LANES):
            sl = pl.ds(cb * LANES, LANES)
            rows_v[r, sl] = rows_v[r, sl] * a
        return c2

      lax.fori_loop(0, ch // LANES, rbody, 0)
      pltpu.make_async_copy(src_hbm.at[pl.ds(base_w + c * ch, ch)],
                            srcs[b], sems_i[b]).wait()
      pltpu.async_copy(rows_v, acc.at[srcs[b]], sems_s[b], add=True)

    start(0, 0)

    def body(g, carry):
      c0 = 2 * g

      @pl.when(g > 0)
      def _():
        wait_scatters(1)

      start(c0 + 1, 1)
      process(c0, 0)

      @pl.when(g < npair - 1)
      def _():
        wait_scatters(0)
        start(c0 + 2, 0)

      process(c0 + 1, 1)
      return carry

    lax.fori_loop(0, npair, body, 0)
    if nchunk % 2:
      wait_scatters(0)
      start(nchunk - 1, 0)
      process(nchunk - 1, 0)
    wait_scatters(0)
    wait_scatters(1)
    plsc.subcore_barrier()

    @pl.when(sid < n_nodes // rpt)
    def _():
      pltpu.sync_copy(acc.at[pl.ds(sid * rpt, rpt)],
                      out_hbm.at[cid, pl.ds(sid * rpt, rpt)])

  return k(x, src, dst, alpha, zeros)


def kernel(x, edge_index, edge_attr, W_h, b_h, W_n, b_n, w_e, W_ft, b_ft):
  n, d = x.shape
  e = edge_index.shape[1]
  src = edge_index[0]
  dst = edge_index[1]

  wa = W_n[:, :d].T
  wb = W_n[:, d:].T
  nb = 2000
  xa, xb = pl.pallas_call(
      _pre_body,
      grid=(n // nb,),
      in_specs=[
          pl.BlockSpec((nb, d), lambda i: (i, 0)),
          pl.BlockSpec(wa.shape, lambda i: (0, 0)),
          pl.BlockSpec(wb.shape, lambda i: (0, 0)),
          pl.BlockSpec((1, d), lambda i: (0, 0)),
      ],
      out_specs=[
          pl.BlockSpec((nb, d), lambda i: (i, 0)),
          pl.BlockSpec((nb, d), lambda i: (i, 0)),
      ],
      out_shape=[
          jax.ShapeDtypeStruct((n, d), jnp.float32),
          jax.ShapeDtypeStruct((n, d), jnp.float32),
      ],
  )(x, wa, wb, b_n.reshape(1, d))

  g = _sc_gather_add(xa, xb, src, dst, n_edges=e, d=d)

  be = 16000
  grid_e = e // be
  ed = edge_attr.shape[1]
  scores, gmax = pl.pallas_call(
      _score_body,
      grid=(grid_e,),
      in_specs=[
          pl.BlockSpec((be, d), lambda i: (i, 0)),
          pl.BlockSpec((ed, be), lambda i: (0, i)),
          pl.BlockSpec((ed, d), lambda i: (0, 0)),
          pl.BlockSpec((1, d), lambda i: (0, 0)),
          pl.BlockSpec((1, d), lambda i: (0, 0)),
      ],
      out_specs=[
          pl.BlockSpec((e,), lambda i: (0,)),
          pl.BlockSpec((1, 1), lambda i: (0, 0)),
      ],
      out_shape=[
          jax.ShapeDtypeStruct((e,), jnp.float32),
          jax.ShapeDtypeStruct((1, 1), jnp.float32),
      ],
  )(g, edge_attr.T, W_h.T, b_h.reshape(1, d), w_e.reshape(1, d))

  alpha = pl.pallas_call(
      _softmax_body,
      in_specs=[
          pl.BlockSpec((e,), lambda: (0,)),
          pl.BlockSpec((1, 1), lambda: (0, 0)),
      ],
      out_specs=pl.BlockSpec((e,), lambda: (0,)),
      out_shape=jax.ShapeDtypeStruct((e,), jnp.float32),
  )(scores, gmax)

  zeros = jnp.zeros((n, d), jnp.float32)
  partials = _sc_scatter(x, src, dst, alpha, zeros,
                         n_nodes=n, n_edges=e, d=d)

  out = pl.pallas_call(
      _final_body,
      grid=(n // nb,),
      in_specs=[
          pl.BlockSpec((nb, d), lambda i: (i, 0)),
          pl.BlockSpec((nb, d), lambda i: (i, 0)),
          pl.BlockSpec((nb, d), lambda i: (i, 0)),
          pl.BlockSpec((d, d), lambda i: (0, 0)),
          pl.BlockSpec((1, d), lambda i: (0, 0)),
      ],
      out_specs=pl.BlockSpec((nb, d), lambda i: (i, 0)),
      out_shape=jax.ShapeDtypeStruct((n, d), jnp.float32),
  )(x, partials[0], partials[1], W_ft.T, b_ft.reshape(1, d))

  return (out, alpha)


# speedup vs baseline: 6.7354x; 1.2394x over previous
import functools

import jax
import jax.numpy as jnp
from jax import lax
from jax.experimental import pallas as pl
from jax.experimental.pallas import tpu as pltpu
from jax.experimental.pallas import tpu_sc as plsc

NC = 2
NS = 16
LANES = 16


def _pre_body(x_ref, wa_ref, wb_ref, bn_ref, xa_ref, xb_ref):
  xblk = x_ref[...]
  xa_ref[...] = jnp.dot(xblk, wa_ref[...], preferred_element_type=jnp.float32)
  xb_ref[...] = (
      jnp.dot(xblk, wb_ref[...], preferred_element_type=jnp.float32)
      + bn_ref[...]
  )


def _score_body(g_ref, ea_ref, wh_ref, bh_ref, we_ref, s_ref, m_ref):
  i = pl.program_id(0)
  be = g_ref.shape[0]
  d = g_ref.shape[1]
  h = (
      lax.dot_general(ea_ref[...], wh_ref[...],
                      dimension_numbers=(((0,), (0,)), ((), ())),
                      preferred_element_type=jnp.float32)
      + bh_ref[...]
      + g_ref[...]
  )
  h = jnp.maximum(h, 0.0)
  we = we_ref[...]
  grps = be // 128
  rows = [
      lax.dot_general(we, lax.slice(h, (r * 128, 0), ((r + 1) * 128, d)),
                      dimension_numbers=(((1,), (1,)), ((), ())),
                      preferred_element_type=jnp.float32)
      for r in range(grps)
  ]
  s2 = jnp.concatenate(rows, axis=0)
  s_ref[pl.ds(i * grps, grps), :] = s2

  @pl.when(i == 0)
  def _():
    m_ref[...] = jnp.full((1, 1), -jnp.inf, jnp.float32)

  m_ref[...] = jnp.maximum(m_ref[...], jnp.max(s2))


def _softmax_body(s_ref, m_ref, a_ref):
  ex = jnp.exp(s_ref[...] - m_ref[0, 0])
  a_ref[...] = ex / jnp.sum(ex)


def _final_body(x_ref, p0_ref, p1_ref, wft_ref, bft_ref, o_ref):
  h = (x_ref[...] - p0_ref[...].astype(jnp.float32)
       - p1_ref[...].astype(jnp.float32))
  o_ref[...] = (
      h
      + jnp.dot(h, wft_ref[...], preferred_element_type=jnp.float32)
      + bft_ref[...]
  )


def _sc_gather_add(xa, xb, src, dst, *, n_edges, d):
  nw = NC * NS
  epw = n_edges // nw
  ch = 200
  nchunk = epw // ch
  npair = nchunk // 2
  mesh = plsc.VectorSubcoreMesh(
      core_axis_name="c", subcore_axis_name="s",
      num_cores=NC, num_subcores=NS)

  @functools.partial(
      pl.kernel,
      out_type=jax.ShapeDtypeStruct((n_edges, d), jnp.float32),
      mesh=mesh,
      scratch_types=[
          pltpu.VMEM((epw,), jnp.int32),
          pltpu.VMEM((epw,), jnp.int32),
          pltpu.VMEM((ch, d), jnp.float32),
          pltpu.VMEM((ch, d), jnp.float32),
          pltpu.VMEM((ch, d), jnp.float32),
          pltpu.VMEM((ch, d), jnp.float32),
          pltpu.SemaphoreType.DMA,
          pltpu.SemaphoreType.DMA,
          pltpu.SemaphoreType.DMA,
          pltpu.SemaphoreType.DMA,
      ],
  )
  def k(xa_hbm, xb_hbm, src_hbm, dst_hbm, g_hbm,
        src_f, dst_f, ra0, ra1, rb0, rb1, sa0, sa1, sb0, sb1):
    wid = lax.axis_index("s") * NC + lax.axis_index("c")
    base_w = wid * epw
    ras = (ra0, ra1)
    rbs = (rb0, rb1)
    sems_a = (sa0, sa1)
    sems_b = (sb0, sb1)

    pltpu.sync_copy(src_hbm.at[pl.ds(base_w, epw)], src_f)
    pltpu.sync_copy(dst_hbm.at[pl.ds(base_w, epw)], dst_f)

    def start(c, b):
      pltpu.async_copy(xa_hbm.at[src_f.at[pl.ds(c * ch, ch)]],
                       ras[b], sems_a[b])
      pltpu.async_copy(xb_hbm.at[dst_f.at[pl.ds(c * ch, ch)]],
                       rbs[b], sems_b[b])

    def process(c, b):
      pltpu.make_async_copy(xa_hbm.at[src_f.at[pl.ds(c * ch, ch)]],
                            ras[b], sems_a[b]).wait()
      pltpu.make_async_copy(xb_hbm.at[dst_f.at[pl.ds(c * ch, ch)]],
                            rbs[b], sems_b[b]).wait()
      ra, rb = ras[b], rbs[b]

      def rbody(q, c2):
        for rr in range(4):
          r = q * 4 + rr
          for cb_ in range(d // LANES):
            sl = pl.ds(cb_ * LANES, LANES)
            ra[r, sl] = ra[r, sl] + rb[r, sl]
        return c2

      lax.fori_loop(0, ch // 4, rbody, 0)
      pltpu.sync_copy(ra, g_hbm.at[pl.ds(base_w + c * ch, ch)])

    start(0, 0)

    def body(g, carry):
      c0 = 2 * g
      start(c0 + 1, 1)
      process(c0, 0)

      @pl.when(g < npair - 1)
      def _():
        start(c0 + 2, 0)

      process(c0 + 1, 1)
      return carry

    lax.fori_loop(0, npair, body, 0)

  return k(xa, xb, src, dst)


def _sc_scatter(x, src, dst, alpha, zeros, *, n_nodes, n_edges, d):
  nw = NC * NS
  epw = n_edges // nw
  ch = 80
  nchunk = epw // ch
  npair = nchunk // 2
  rpt = 1000
  mesh = plsc.VectorSubcoreMesh(
      core_axis_name="c", subcore_axis_name="s",
      num_cores=NC, num_subcores=NS)

  @functools.partial(
      pl.kernel,
      out_type=jax.ShapeDtypeStruct((NC, n_nodes, d), jnp.float32),
      mesh=mesh,
      scratch_types=[
          pltpu.VMEM((epw,), jnp.int32),
          pltpu.VMEM((epw,), jnp.float32),
          pltpu.VMEM((ch,), jnp.int32),
          pltpu.VMEM((ch,), jnp.int32),
          pltpu.VMEM((ch, d), jnp.float32),
          pltpu.VMEM((ch, d), jnp.float32),
          pltpu.SemaphoreType.DMA,
          pltpu.SemaphoreType.DMA,
          pltpu.SemaphoreType.DMA,
          pltpu.SemaphoreType.DMA,
          pltpu.SemaphoreType.DMA,
          pltpu.SemaphoreType.DMA,
          pltpu.VMEM_SHARED((n_nodes, d), jnp.float32),
      ],
  )
  def k(x_hbm, src_hbm, dst_hbm, alpha_hbm, zeros_hbm, out_hbm,
        dst_f, alpha_f, sv0, sv1, rv0, rv1,
        sg0, sg1, ss0, ss1, si0, si1, acc):
    cid = lax.axis_index("c")
    sid = lax.axis_index("s")
    srcs = (sv0, sv1)
    rows = (rv0, rv1)
    sems_g = (sg0, sg1)
    sems_s = (ss0, ss1)
    sems_i = (si0, si1)

    @pl.when(sid == 0)
    def _():
      pltpu.sync_copy(zeros_hbm, acc)

    plsc.subcore_barrier()

    wid = sid * NC + cid
    base_w = wid * epw

    pltpu.sync_copy(dst_hbm.at[pl.ds(base_w, epw)], dst_f)
    pltpu.sync_copy(alpha_hbm.at[pl.ds(base_w, epw)], alpha_f)

    def start(c, b):
      pltpu.async_copy(src_hbm.at[pl.ds(base_w + c * ch, ch)],
                       srcs[b], sems_i[b])
      pltpu.async_copy(x_hbm.at[dst_f.at[pl.ds(c * ch, ch)]],
                       rows[b], sems_g[b])

    def wait_scatters(b):
      pltpu.make_async_copy(rows[b], acc.at[srcs[b]], sems_s[b]).wait()

    def process(c, b):
      pltpu.make_async_copy(x_hbm.at[dst_f.at[pl.ds(c * ch, ch)]],
                            rows[b], sems_g[b]).wait()
      rows_v = rows[b]

      def rbody(grp, c2):
        a16 = alpha_f[pl.ds(c * ch + grp * LANES, LANES)]
        for rr in range(LANES):
          r = grp * LANES + rr
          a = a16[rr]
          for cb in range(d // LANES):
            sl = pl.ds(cb * LANES, LANES)
            rows_v[r, sl] = rows_v[r, sl] * a
        return c2

      lax.fori_loop(0, ch // LANES, rbody, 0)
      pltpu.make_async_copy(src_hbm.at[pl.ds(base_w + c * ch, ch)],
                            srcs[b], sems_i[b]).wait()
      pltpu.async_copy(rows_v, acc.at[srcs[b]], sems_s[b], add=True)

    start(0, 0)

    def body(g, carry):
      c0 = 2 * g

      @pl.when(g > 0)
      def _():
        wait_scatters(1)

      start(c0 + 1, 1)
      process(c0, 0)

      @pl.when(g < npair - 1)
      def _():
        wait_scatters(0)
        start(c0 + 2, 0)

      process(c0 + 1, 1)
      return carry

    lax.fori_loop(0, npair, body, 0)
    if nchunk % 2:
      wait_scatters(0)
      start(nchunk - 1, 0)
      process(nchunk - 1, 0)
    wait_scatters(0)
    wait_scatters(1)
    plsc.subcore_barrier()

    @pl.when(sid < n_nodes // rpt)
    def _():
      pltpu.sync_copy(acc.at[pl.ds(sid * rpt, rpt)],
                      out_hbm.at[cid, pl.ds(sid * rpt, rpt)])

  return k(x, src, dst, alpha, zeros)


def kernel(x, edge_index, edge_attr, W_h, b_h, W_n, b_n, w_e, W_ft, b_ft):
  n, d = x.shape
  e = edge_index.shape[1]
  src = edge_index[0]
  dst = edge_index[1]

  wa = W_n[:, :d].T
  wb = W_n[:, d:].T
  nb = 2000
  xa, xb = pl.pallas_call(
      _pre_body,
      grid=(n // nb,),
      in_specs=[
          pl.BlockSpec((nb, d), lambda i: (i, 0)),
          pl.BlockSpec(wa.shape, lambda i: (0, 0)),
          pl.BlockSpec(wb.shape, lambda i: (0, 0)),
          pl.BlockSpec((1, d), lambda i: (0, 0)),
      ],
      out_specs=[
          pl.BlockSpec((nb, d), lambda i: (i, 0)),
          pl.BlockSpec((nb, d), lambda i: (i, 0)),
      ],
      out_shape=[
          jax.ShapeDtypeStruct((n, d), jnp.float32),
          jax.ShapeDtypeStruct((n, d), jnp.float32),
      ],
  )(x, wa, wb, b_n.reshape(1, d))

  g = _sc_gather_add(xa, xb, src, dst, n_edges=e, d=d)

  be = 16000
  grid_e = e // be
  ed = edge_attr.shape[1]
  scores, gmax = pl.pallas_call(
      _score_body,
      grid=(grid_e,),
      in_specs=[
          pl.BlockSpec((be, d), lambda i: (i, 0)),
          pl.BlockSpec((ed, be), lambda i: (0, i)),
          pl.BlockSpec((ed, d), lambda i: (0, 0)),
          pl.BlockSpec((1, d), lambda i: (0, 0)),
          pl.BlockSpec((1, d), lambda i: (0, 0)),
      ],
      out_specs=[
          pl.BlockSpec((e // 128, 128), lambda i: (0, 0)),
          pl.BlockSpec((1, 1), lambda i: (0, 0)),
      ],
      out_shape=[
          jax.ShapeDtypeStruct((e // 128, 128), jnp.float32),
          jax.ShapeDtypeStruct((1, 1), jnp.float32),
      ],
  )(g, edge_attr.T, W_h.T, b_h.reshape(1, d), w_e.reshape(1, d))

  alpha2d = pl.pallas_call(
      _softmax_body,
      in_specs=[
          pl.BlockSpec((e // 128, 128), lambda: (0, 0)),
          pl.BlockSpec((1, 1), lambda: (0, 0)),
      ],
      out_specs=pl.BlockSpec((e // 128, 128), lambda: (0, 0)),
      out_shape=jax.ShapeDtypeStruct((e // 128, 128), jnp.float32),
  )(scores, gmax)
  alpha = alpha2d.reshape(e)

  zeros = jnp.zeros((n, d), jnp.float32)
  partials = _sc_scatter(x, src, dst, alpha, zeros,
                         n_nodes=n, n_edges=e, d=d)

  out = pl.pallas_call(
      _final_body,
      grid=(n // nb,),
      in_specs=[
          pl.BlockSpec((nb, d), lambda i: (i, 0)),
          pl.BlockSpec((nb, d), lambda i: (i, 0)),
          pl.BlockSpec((nb, d), lambda i: (i, 0)),
          pl.BlockSpec((d, d), lambda i: (0, 0)),
          pl.BlockSpec((1, d), lambda i: (0, 0)),
      ],
      out_specs=pl.BlockSpec((nb, d), lambda i: (i, 0)),
      out_shape=jax.ShapeDtypeStruct((n, d), jnp.float32),
  )(x, partials[0], partials[1], W_ft.T, b_ft.reshape(1, d))

  return (out, alpha)
